# baseline (speedup 1.0000x reference)
import dataclasses
import functools

import jax
import jax.numpy as jnp
from jax import lax
from jax.experimental import pallas as pl
from jax.experimental.pallas import tpu as pltpu
from jax.experimental.pallas import tpu_sc as plsc

PAD = 0
NUM_CORES = 2
NUM_SUBCORES = 16
NUM_WORKERS = NUM_CORES * NUM_SUBCORES
LANES = 16
IDX_PER_DMA = 128
GATHERS_PER_HALF = 2
HALF = IDX_PER_DMA * GATHERS_PER_HALF
SUPER = 2 * HALF
IDX_ROWS = SUPER // IDX_PER_DMA
PDIM = 128


def _prep_table(table_t):
    d, v = table_t.shape
    blk = 4096
    grid = (v + blk - 1) // blk

    def body(x_ref, o_ref):
        x = x_ref[...]
        y = x.T
        o_ref[...] = jnp.concatenate(
            [y, jnp.zeros((blk, PDIM - d), jnp.float32)], axis=1
        )

    return pl.pallas_call(
        body,
        grid=(grid,),
        in_specs=[pl.BlockSpec((d, blk), lambda i: (0, i))],
        out_specs=pl.BlockSpec((blk, PDIM), lambda i: (i, 0)),
        out_shape=jax.ShapeDtypeStruct((v, PDIM), jnp.float32),
        compiler_params=pltpu.CompilerParams(
            dimension_semantics=("parallel",),
        ),
    )(table_t)


def _format_out(g, b, l, d):
    bblk = 256
    grid = b // bblk

    def body(x_ref, o_ref):
        x = x_ref[...].reshape(bblk, l, PDIM)
        for ll in range(l):
            o_ref[ll] = x[:, ll, :d].T

    return pl.pallas_call(
        body,
        grid=(grid,),
        in_specs=[pl.BlockSpec((bblk * l, PDIM), lambda i: (i, 0))],
        out_specs=pl.BlockSpec((l, d, bblk), lambda i: (0, 0, i)),
        out_shape=jax.ShapeDtypeStruct((l, d, b), jnp.float32),
        compiler_params=pltpu.CompilerParams(
            dimension_semantics=("parallel",),
        ),
    )(g)


def _make_gather(n_rows: int):
    assert n_rows % (NUM_WORKERS * SUPER) == 0
    per_worker = n_rows // NUM_WORKERS
    steps = per_worker // SUPER
    mesh = plsc.VectorSubcoreMesh(core_axis_name="c", subcore_axis_name="s")
    cp = pltpu.CompilerParams()
    if "needs_layout_passes" in pltpu.CompilerParams.__dataclass_fields__:
        cp = dataclasses.replace(cp, needs_layout_passes=False)
    if "use_tc_tiling_on_sc" in pltpu.CompilerParams.__dataclass_fields__:
        cp = dataclasses.replace(cp, use_tc_tiling_on_sc=False)

    @functools.partial(
        pl.kernel,
        mesh=mesh,
        compiler_params=cp,
        out_type=jax.ShapeDtypeStruct((n_rows, PDIM), jnp.float32),
        scratch_types=[
            pltpu.VMEM((IDX_ROWS, IDX_PER_DMA), jnp.int32),
            pltpu.VMEM((HALF, PDIM), jnp.float32),
            pltpu.VMEM((HALF, PDIM), jnp.float32),
            pltpu.SemaphoreType.DMA,
            pltpu.SemaphoreType.DMA,
            pltpu.SemaphoreType.DMA,
            pltpu.SemaphoreType.DMA,
        ],
    )
    def gather_kernel(
        idx_hbm, table_hbm, out_hbm, idx_v, rows0, rows1,
        gsem0, gsem1, wsem0, wsem1,
    ):
        wid = lax.axis_index("s") * NUM_CORES + lax.axis_index("c")
        row0 = wid * per_worker

        def fire_gathers(half, rows_v, gsem):
            return [
                pltpu.async_copy(
                    table_hbm.at[idx_v.at[half * GATHERS_PER_HALF + j]],
                    rows_v.at[pl.ds(j * IDX_PER_DMA, IDX_PER_DMA)],
                    gsem,
                )
                for j in range(GATHERS_PER_HALF)
            ]

        def mask_rows(half, rows_v):
            @pl.loop(0, HALF, step=LANES)
            def _(r):
                g = half * HALF + r
                jblk = g // IDX_PER_DMA
                roff = g - jblk * IDX_PER_DMA
                iv = idx_v[jblk, pl.ds(roff, LANES)]

                @pl.when(jnp.min(iv) == PAD)
                def _():
                    @pl.loop(0, LANES)
                    def _(jj):
                        sel = jnp.full((LANES,), roff + jj, jnp.int32)
                        jb = jnp.full((LANES,), jblk, jnp.int32)
                        ival = plsc.load_gather(idx_v, [jb, sel])
                        m = jnp.where(ival == PAD, 0.0, 1.0).astype(
                            jnp.float32
                        )
                        row = r + jj
                        for col in range(0, PDIM, LANES):
                            cur = rows_v[row, pl.ds(col, LANES)]
                            rows_v[row, pl.ds(col, LANES)] = cur * m

        @pl.loop(0, steps)
        def _(k):
            base = row0 + k * SUPER
            idx_row = pl.multiple_of(base // IDX_PER_DMA, IDX_ROWS)
            pltpu.sync_copy(idx_hbm.at[pl.ds(idx_row, IDX_ROWS)], idx_v)

            @pl.when(k > 0)
            def _():
                pltpu.make_async_copy(
                    rows0, out_hbm.at[pl.ds(row0, HALF)], wsem0
                ).wait()

            g0 = fire_gathers(0, rows0, gsem0)

            @pl.when(k > 0)
            def _():
                pltpu.make_async_copy(
                    rows1, out_hbm.at[pl.ds(row0, HALF)], wsem1
                ).wait()

            g1 = fire_gathers(1, rows1, gsem1)

            for c in g0:
                c.wait()
            mask_rows(0, rows0)
            pltpu.async_copy(rows0, out_hbm.at[pl.ds(base, HALF)], wsem0)

            for c in g1:
                c.wait()
            mask_rows(1, rows1)
            pltpu.async_copy(
                rows1, out_hbm.at[pl.ds(base + HALF, HALF)], wsem1
            )

        pltpu.make_async_copy(
            rows0, out_hbm.at[pl.ds(row0, HALF)], wsem0
        ).wait()
        pltpu.make_async_copy(
            rows1, out_hbm.at[pl.ds(row0, HALF)], wsem1
        ).wait()

    return gather_kernel


def kernel(tokens, table):
    b, l = tokens.shape
    _, d = table.shape
    n = b * l
    idx2d = tokens.reshape(n // IDX_PER_DMA, IDX_PER_DMA)
    tbl128 = _prep_table(table.T)
    g = _make_gather(n)(idx2d, tbl128)
    out_t = _format_out(g, b, l, d)
    feats = jnp.transpose(out_t, (2, 0, 1))
    padding_masks = (tokens == PAD)[:, None, None, :]
    sequential_masks = jnp.triu(jnp.ones((l, l), dtype=jnp.bool_), k=1)
    return feats, padding_masks, sequential_masks


# speedup vs baseline: 1.0708x; 1.0708x over previous
import dataclasses
import functools

import jax
import jax.numpy as jnp
from jax import lax
from jax.experimental import pallas as pl
from jax.experimental.pallas import tpu as pltpu
from jax.experimental.pallas import tpu_sc as plsc

PAD = 0
NUM_CORES = 2
NUM_SUBCORES = 16
NUM_WORKERS = NUM_CORES * NUM_SUBCORES
LANES = 16
IDX_PER_DMA = 128
GATHERS_PER_HALF = 2
HALF = IDX_PER_DMA * GATHERS_PER_HALF
SUPER = 2 * HALF
IDX_ROWS = SUPER // IDX_PER_DMA
PDIM = 128


def _prep_table(table_t):
    d, v = table_t.shape
    blk = 8192
    grid = (v + blk - 1) // blk

    def body(x_ref, o_ref):
        o_ref[:, :d] = x_ref[...].T

    return pl.pallas_call(
        body,
        grid=(grid,),
        in_specs=[pl.BlockSpec((d, blk), lambda i: (0, i))],
        out_specs=pl.BlockSpec((blk, PDIM), lambda i: (i, 0)),
        out_shape=jax.ShapeDtypeStruct((v, PDIM), jnp.float32),
        compiler_params=pltpu.CompilerParams(
            dimension_semantics=("parallel",),
        ),
    )(table_t)


def _format_out(g, b, l, d):
    bblk = 512
    grid = b // bblk

    def body(x_ref, o_ref):
        x = x_ref[...].reshape(bblk, l, PDIM)
        for ll in range(l):
            o_ref[ll] = x[:, ll, :d].T

    return pl.pallas_call(
        body,
        grid=(grid,),
        in_specs=[pl.BlockSpec((bblk * l, PDIM), lambda i: (i, 0))],
        out_specs=pl.BlockSpec((l, d, bblk), lambda i: (0, 0, i)),
        out_shape=jax.ShapeDtypeStruct((l, d, b), jnp.float32),
        compiler_params=pltpu.CompilerParams(
            dimension_semantics=("parallel",),
        ),
    )(g)


def _make_gather(n_rows: int):
    assert n_rows % (NUM_WORKERS * SUPER) == 0
    per_worker = n_rows // NUM_WORKERS
    steps = per_worker // SUPER
    mesh = plsc.VectorSubcoreMesh(core_axis_name="c", subcore_axis_name="s")
    cp = pltpu.CompilerParams()
    if "needs_layout_passes" in pltpu.CompilerParams.__dataclass_fields__:
        cp = dataclasses.replace(cp, needs_layout_passes=False)
    if "use_tc_tiling_on_sc" in pltpu.CompilerParams.__dataclass_fields__:
        cp = dataclasses.replace(cp, use_tc_tiling_on_sc=False)

    @functools.partial(
        pl.kernel,
        mesh=mesh,
        compiler_params=cp,
        out_type=jax.ShapeDtypeStruct((n_rows, PDIM), jnp.float32),
        scratch_types=[
            pltpu.VMEM((IDX_ROWS, IDX_PER_DMA), jnp.int32),
            pltpu.VMEM((HALF, PDIM), jnp.float32),
            pltpu.VMEM((HALF, PDIM), jnp.float32),
            pltpu.SemaphoreType.DMA,
            pltpu.SemaphoreType.DMA,
            pltpu.SemaphoreType.DMA,
            pltpu.SemaphoreType.DMA,
        ],
    )
    def gather_kernel(
        idx_hbm, table_hbm, out_hbm, idx_v, rows0, rows1,
        gsem0, gsem1, wsem0, wsem1,
    ):
        wid = lax.axis_index("s") * NUM_CORES + lax.axis_index("c")
        row0 = wid * per_worker

        def fire_gathers(half, rows_v, gsem):
            return [
                pltpu.async_copy(
                    table_hbm.at[idx_v.at[half * GATHERS_PER_HALF + j]],
                    rows_v.at[pl.ds(j * IDX_PER_DMA, IDX_PER_DMA)],
                    gsem,
                )
                for j in range(GATHERS_PER_HALF)
            ]

        def mask_rows(half, rows_v):
            @pl.loop(0, HALF, step=LANES)
            def _(r):
                g = half * HALF + r
                jblk = g // IDX_PER_DMA
                roff = g - jblk * IDX_PER_DMA
                iv = idx_v[jblk, pl.ds(roff, LANES)]

                @pl.when(jnp.min(iv) == PAD)
                def _():
                    @pl.loop(0, LANES)
                    def _(jj):
                        sel = jnp.full((LANES,), roff + jj, jnp.int32)
                        jb = jnp.full((LANES,), jblk, jnp.int32)
                        ival = plsc.load_gather(idx_v, [jb, sel])
                        m = jnp.where(ival == PAD, 0.0, 1.0).astype(
                            jnp.float32
                        )
                        row = r + jj
                        for col in range(0, PDIM, LANES):
                            cur = rows_v[row, pl.ds(col, LANES)]
                            rows_v[row, pl.ds(col, LANES)] = cur * m

        @pl.loop(0, steps)
        def _(k):
            base = row0 + k * SUPER
            idx_row = pl.multiple_of(base // IDX_PER_DMA, IDX_ROWS)
            pltpu.sync_copy(idx_hbm.at[pl.ds(idx_row, IDX_ROWS)], idx_v)

            @pl.when(k > 0)
            def _():
                pltpu.make_async_copy(
                    rows0, out_hbm.at[pl.ds(row0, HALF)], wsem0
                ).wait()

            g0 = fire_gathers(0, rows0, gsem0)

            @pl.when(k > 0)
            def _():
                pltpu.make_async_copy(
                    rows1, out_hbm.at[pl.ds(row0, HALF)], wsem1
                ).wait()

            g1 = fire_gathers(1, rows1, gsem1)

            for c in g0:
                c.wait()
            mask_rows(0, rows0)
            pltpu.async_copy(rows0, out_hbm.at[pl.ds(base, HALF)], wsem0)

            for c in g1:
                c.wait()
            mask_rows(1, rows1)
            pltpu.async_copy(
                rows1, out_hbm.at[pl.ds(base + HALF, HALF)], wsem1
            )

        pltpu.make_async_copy(
            rows0, out_hbm.at[pl.ds(row0, HALF)], wsem0
        ).wait()
        pltpu.make_async_copy(
            rows1, out_hbm.at[pl.ds(row0, HALF)], wsem1
        ).wait()

    return gather_kernel


def kernel(tokens, table):
    b, l = tokens.shape
    _, d = table.shape
    n = b * l
    idx2d = tokens.reshape(n // IDX_PER_DMA, IDX_PER_DMA)
    tbl128 = _prep_table(table.T)
    g = _make_gather(n)(idx2d, tbl128)
    out_t = _format_out(g, b, l, d)
    feats = jnp.transpose(out_t, (2, 0, 1))
    padding_masks = (tokens == PAD)[:, None, None, :]
    sequential_masks = jnp.triu(jnp.ones((l, l), dtype=jnp.bool_), k=1)
    return feats, padding_masks, sequential_masks
